# baseline (speedup 1.0000x reference)
import functools

import jax
import jax.numpy as jnp
from jax import lax
from jax.experimental import pallas as pl
from jax.experimental.pallas import tpu as pltpu
from jax.experimental.pallas import tpu_sc as plsc

N_NODES = 10000
N_EDGES = 320000
LATENT = 128
HIDDEN = 256

_LOG2 = 0.6931471805599453


def _ssp(x):
    return jnp.maximum(x, 0.0) + jnp.log1p(jnp.exp(-jnp.abs(x))) - _LOG2


def _ln(x):
    m = jnp.mean(x, axis=-1, keepdims=True)
    v = jnp.mean((x - m) ** 2, axis=-1, keepdims=True)
    return (x - m) / jnp.sqrt(v + 1e-5)


def _dot(a, b):
    return jnp.dot(a, b, preferred_element_type=jnp.float32)


def _full(shape=None):
    return pl.BlockSpec(shape, lambda i: (0, 0)) if shape else pl.BlockSpec(
        memory_space=pltpu.ANY)


def _wspec(shape):
    return pl.BlockSpec(shape, lambda i: (0, 0))


def _mlp_ln_body(x_ref, w1, b1, w2, b2, w3, b3, o_ref):
    h = _ssp(_dot(x_ref[...], w1[...]) + b1[...])
    h = _ssp(_dot(h, w2[...]) + b2[...])
    h = _dot(h, w3[...]) + b3[...]
    o_ref[...] = _ln(h)


def _mlp_ln(x, params, tile):
    (w1, b1), (w2, b2), (w3, b3) = params
    n, din = x.shape
    dout = w3.shape[1]
    grid = (n + tile - 1) // tile
    return pl.pallas_call(
        _mlp_ln_body,
        grid=(grid,),
        in_specs=[
            pl.BlockSpec((tile, din), lambda i: (i, 0)),
            _wspec(w1.shape), _wspec((1, w1.shape[1])),
            _wspec(w2.shape), _wspec((1, w2.shape[1])),
            _wspec(w3.shape), _wspec((1, w3.shape[1])),
        ],
        out_specs=pl.BlockSpec((tile, dout), lambda i: (i, 0)),
        out_shape=jax.ShapeDtypeStruct((n, dout), jnp.float32),
    )(x, w1, b1.reshape(1, -1), w2, b2.reshape(1, -1), w3, b3.reshape(1, -1))


def _dec_body(x_ref, w1, b1, w2, b2, w3, b3, o_ref):
    h = _ssp(_dot(x_ref[...], w1[...]) + b1[...])
    h = _ssp(_dot(h, w2[...]) + b2[...])
    o_ref[...] = _dot(h, w3[...]) + b3[...]


def _decode(x, params, tile):
    (w1, b1), (w2, b2), (w3, b3) = params
    n, din = x.shape
    dout = w3.shape[1]
    w3p = jnp.pad(w3, ((0, 0), (0, 8 - dout)))
    b3p = jnp.pad(b3, (0, 8 - dout))
    grid = (n + tile - 1) // tile
    out = pl.pallas_call(
        _dec_body,
        grid=(grid,),
        in_specs=[
            pl.BlockSpec((tile, din), lambda i: (i, 0)),
            _wspec(w1.shape), _wspec((1, w1.shape[1])),
            _wspec(w2.shape), _wspec((1, w2.shape[1])),
            _wspec((din * 0 + w2.shape[1], 8)), _wspec((1, 8)),
        ],
        out_specs=pl.BlockSpec((tile, 8), lambda i: (i, 0)),
        out_shape=jax.ShapeDtypeStruct((n, 8), jnp.float32),
    )(x, w1, b1.reshape(1, -1), w2, b2.reshape(1, -1), w3p, b3p.reshape(1, -1))
    return out[:, :dout]


def _edge_body(e_ref, s_ref, r_ref, c_ref, gw, gb, w1e, w1s, w1r, b1, w2, b2,
               w3, b3, ne_ref, ms_ref, mr_ref):
    e = e_ref[...]
    h = _dot(e, w1e[...]) + _dot(s_ref[...], w1s[...]) + _dot(
        r_ref[...], w1r[...]) + b1[...]
    h = _ssp(h)
    h = _ssp(_dot(h, w2[...]) + b2[...])
    u = _ln(_dot(h, w3[...]) + b3[...])
    c = c_ref[...]
    g_r = jax.nn.sigmoid(
        jnp.sum(e * gw[0:1, :], axis=1, keepdims=True) + gb[0]) * c
    g_s = jax.nn.sigmoid(
        jnp.sum(e * gw[1:2, :], axis=1, keepdims=True) + gb[1]) * c
    ne_ref[...] = e + u
    ms_ref[...] = u * g_s
    mr_ref[...] = u * g_r


def _edge_step(edges, sent, recv, cutoff, p, tile):
    (w1, b1), (w2, b2), (w3, b3) = p["edge_mlp"]
    gw = jnp.concatenate([p["rw"], p["sw"]], axis=1).T
    gb = jnp.concatenate([p["rb"], p["sb"]])
    n = edges.shape[0]
    grid = (n + tile - 1) // tile
    row = lambda i: (i, 0)
    outs = pl.pallas_call(
        _edge_body,
        grid=(grid,),
        in_specs=[
            pl.BlockSpec((tile, LATENT), row),
            pl.BlockSpec((tile, LATENT), row),
            pl.BlockSpec((tile, LATENT), row),
            pl.BlockSpec((tile, 1), row),
            _wspec((2, LATENT)),
            pl.BlockSpec(memory_space=pltpu.SMEM),
            _wspec((LATENT, HIDDEN)), _wspec((LATENT, HIDDEN)),
            _wspec((LATENT, HIDDEN)), _wspec((1, HIDDEN)),
            _wspec((HIDDEN, HIDDEN)), _wspec((1, HIDDEN)),
            _wspec((HIDDEN, LATENT)), _wspec((1, LATENT)),
        ],
        out_specs=[
            pl.BlockSpec((tile, LATENT), row),
            pl.BlockSpec((tile, LATENT), row),
            pl.BlockSpec((tile, LATENT), row),
        ],
        out_shape=[jax.ShapeDtypeStruct((n, LATENT), jnp.float32)] * 3,
    )(edges, sent, recv, cutoff, gw, gb,
      w1[:LATENT], w1[LATENT:2 * LATENT], w1[2 * LATENT:], b1.reshape(1, -1),
      w2, b2.reshape(1, -1), w3, b3.reshape(1, -1))
    return outs


def _node_body(x_ref, ra0_ref, ra1_ref, sa0_ref, sa1_ref, w1n, w1r, w1s, b1,
               w2, b2, w3, b3, o_ref):
    x = x_ref[...]
    ra = ra0_ref[...] + ra1_ref[...]
    sa = sa0_ref[...] + sa1_ref[...]
    h = _dot(x, w1n[...]) + _dot(ra, w1r[...]) + _dot(sa, w1s[...]) + b1[...]
    h = _ssp(h)
    h = _ssp(_dot(h, w2[...]) + b2[...])
    u = _ln(_dot(h, w3[...]) + b3[...])
    o_ref[...] = x + u


def _node_step(nodes, aggs, p, tile):
    (w1, b1), (w2, b2), (w3, b3) = p["node_mlp"]
    (ra0, sa0), (ra1, sa1) = aggs
    n = nodes.shape[0]
    grid = (n + tile - 1) // tile
    row = lambda i: (i, 0)
    return pl.pallas_call(
        _node_body,
        grid=(grid,),
        in_specs=[
            pl.BlockSpec((tile, LATENT), row),
            pl.BlockSpec((tile, LATENT), row),
            pl.BlockSpec((tile, LATENT), row),
            pl.BlockSpec((tile, LATENT), row),
            pl.BlockSpec((tile, LATENT), row),
            _wspec((LATENT, HIDDEN)), _wspec((LATENT, HIDDEN)),
            _wspec((LATENT, HIDDEN)), _wspec((1, HIDDEN)),
            _wspec((HIDDEN, HIDDEN)), _wspec((1, HIDDEN)),
            _wspec((HIDDEN, LATENT)), _wspec((1, LATENT)),
        ],
        out_specs=pl.BlockSpec((tile, LATENT), row),
        out_shape=jax.ShapeDtypeStruct((n, LATENT), jnp.float32),
    )(nodes, ra0, ra1, sa0, sa1,
      w1[:LATENT], w1[LATENT:2 * LATENT], w1[2 * LATENT:], b1.reshape(1, -1),
      w2, b2.reshape(1, -1), w3, b3.reshape(1, -1))


_CHUNK = 128
_NSUB = 16


def _sc_gather(nodes, senders, receivers):
    ne = senders.shape[0]
    nchunks = ne // _CHUNK
    maxk = (nchunks + _NSUB - 1) // _NSUB
    mesh = plsc.VectorSubcoreMesh(core_axis_name="c", subcore_axis_name="s")

    @functools.partial(
        pl.kernel,
        out_type=(jax.ShapeDtypeStruct((ne, LATENT), jnp.float32),
                  jax.ShapeDtypeStruct((ne, LATENT), jnp.float32)),
        mesh=mesh,
        scratch_types=[
            pltpu.VMEM((_CHUNK,), jnp.int32),
            pltpu.VMEM((_CHUNK, LATENT), jnp.float32),
        ],
    )
    def k(nodes_hbm, snd_hbm, rcv_hbm, sent_hbm, recv_hbm, idx_v, buf_v):
        core = lax.axis_index("c")
        sub = lax.axis_index("s")

        def do(idx_hbm, out_hbm):
            @pl.loop(0, maxk)
            def _(kk):
                c = sub + kk * _NSUB

                @pl.when(c < nchunks)
                def _():
                    off = pl.multiple_of(c * _CHUNK, 8)
                    pltpu.sync_copy(idx_hbm.at[pl.ds(off, _CHUNK)], idx_v)
                    pltpu.sync_copy(nodes_hbm.at[idx_v], buf_v)
                    pltpu.sync_copy(buf_v, out_hbm.at[pl.ds(off, _CHUNK)])

        @pl.when(core == 0)
        def _():
            do(rcv_hbm, recv_hbm)

        @pl.when(core == 1)
        def _():
            do(snd_hbm, sent_hbm)

    sent, recv = k(nodes, senders, receivers)
    return sent, recv


def _sc_segsum(msg_r, msg_s, receivers, senders, zeros):
    ne = receivers.shape[0]
    nchunks = ne // _CHUNK
    maxk = (nchunks + _NSUB - 1) // _NSUB
    mesh = plsc.VectorSubcoreMesh(core_axis_name="c", subcore_axis_name="s")
    rchunk = 80
    nrch = N_NODES // rchunk
    maxr = (nrch + _NSUB - 1) // _NSUB

    @functools.partial(
        pl.kernel,
        out_type=jax.ShapeDtypeStruct((2, N_NODES, LATENT), jnp.float32),
        mesh=mesh,
        scratch_types=[
            pltpu.VMEM((_CHUNK,), jnp.int32),
            pltpu.VMEM((_CHUNK, LATENT), jnp.float32),
            pltpu.VMEM_SHARED((N_NODES, LATENT), jnp.float32),
        ],
    )
    def k(mr_hbm, ms_hbm, rcv_hbm, snd_hbm, z_hbm, out_hbm, idx_v, buf_v, acc):
        core = lax.axis_index("c")
        sub = lax.axis_index("s")

        @pl.loop(0, maxr)
        def _(kk):
            c = sub + kk * _NSUB

            @pl.when(c < nrch)
            def _():
                off = pl.multiple_of(c * rchunk, 8)
                pltpu.sync_copy(z_hbm.at[pl.ds(off, rchunk)],
                                acc.at[pl.ds(off, rchunk)])

        plsc.subcore_barrier()

        def do(m_hbm, i_hbm):
            @pl.loop(0, maxk)
            def _(kk):
                c = sub + kk * _NSUB

                @pl.when(c < nchunks)
                def _():
                    off = pl.multiple_of(c * _CHUNK, 8)
                    pltpu.sync_copy(i_hbm.at[pl.ds(off, _CHUNK)], idx_v)
                    pltpu.sync_copy(m_hbm.at[pl.ds(off, _CHUNK)], buf_v)
                    pltpu.sync_copy(buf_v, acc.at[idx_v], add=True)

        @pl.when(core == 0)
        def _():
            do(mr_hbm, rcv_hbm)

        @pl.when(core == 1)
        def _():
            do(ms_hbm, snd_hbm)

        plsc.subcore_barrier()

        @pl.loop(0, maxr)
        def _(kk):
            c = sub + kk * _NSUB

            @pl.when(c < nrch)
            def _():
                off = pl.multiple_of(c * rchunk, 8)
                pltpu.sync_copy(acc.at[pl.ds(off, rchunk)],
                                out_hbm.at[core, pl.ds(off, rchunk)])

    out = k(msg_r, msg_s, receivers, senders, zeros)
    return out[0], out[1]


def kernel(node_features, edge_features, senders, receivers, cutoff, params):
    nodes = _mlp_ln(node_features, params["enc_node"], 512)
    zeros = jnp.zeros((N_NODES, LATENT), jnp.float32)
    h = N_EDGES // 2
    sl = (slice(0, h), slice(h, N_EDGES))
    snd = [senders[q] for q in sl]
    rcv = [receivers[q] for q in sl]
    cut = [cutoff[q] for q in sl]
    edges = [_mlp_ln(edge_features[q], params["enc_edge"], 512) for q in sl]
    for s in range(3):
        p = params["gnn"][s]
        new_e, aggs = [], []
        gathered = [_sc_gather(nodes, snd[i], rcv[i]) for i in range(2)]
        for i in range(2):
            sent_i, recv_i = gathered[i]
            ne, ms, mr = _edge_step(edges[i], sent_i, recv_i, cut[i], p, 512)
            new_e.append(ne)
            aggs.append(_sc_segsum(mr, ms, rcv[i], snd[i], zeros))
        nodes = _node_step(nodes, aggs, p, 512)
        edges = new_e
    return _decode(nodes, params["dec"], 512)


# speedup vs baseline: 1.0351x; 1.0351x over previous
import functools

import jax
import jax.numpy as jnp
from jax import lax
from jax.experimental import pallas as pl
from jax.experimental.pallas import tpu as pltpu
from jax.experimental.pallas import tpu_sc as plsc

N_NODES = 10000
N_EDGES = 320000
LATENT = 128
HIDDEN = 256

_LOG2 = 0.6931471805599453


def _ssp(x):
    return jnp.maximum(x, 0.0) + jnp.log1p(jnp.exp(-jnp.abs(x))) - _LOG2


def _ln(x):
    m = jnp.mean(x, axis=-1, keepdims=True)
    v = jnp.mean((x - m) ** 2, axis=-1, keepdims=True)
    return (x - m) / jnp.sqrt(v + 1e-5)


def _dot(a, b):
    return jnp.dot(a, b, preferred_element_type=jnp.float32)


def _full(shape=None):
    return pl.BlockSpec(shape, lambda i: (0, 0)) if shape else pl.BlockSpec(
        memory_space=pltpu.ANY)


def _wspec(shape):
    return pl.BlockSpec(shape, lambda i: (0, 0))


def _mlp_ln_body(x_ref, w1, b1, w2, b2, w3, b3, o_ref):
    h = _ssp(_dot(x_ref[...], w1[...]) + b1[...])
    h = _ssp(_dot(h, w2[...]) + b2[...])
    h = _dot(h, w3[...]) + b3[...]
    o_ref[...] = _ln(h)


def _mlp_ln(x, params, tile):
    (w1, b1), (w2, b2), (w3, b3) = params
    n, din = x.shape
    dout = w3.shape[1]
    grid = (n + tile - 1) // tile
    return pl.pallas_call(
        _mlp_ln_body,
        grid=(grid,),
        in_specs=[
            pl.BlockSpec((tile, din), lambda i: (i, 0)),
            _wspec(w1.shape), _wspec((1, w1.shape[1])),
            _wspec(w2.shape), _wspec((1, w2.shape[1])),
            _wspec(w3.shape), _wspec((1, w3.shape[1])),
        ],
        out_specs=pl.BlockSpec((tile, dout), lambda i: (i, 0)),
        out_shape=jax.ShapeDtypeStruct((n, dout), jnp.float32),
    )(x, w1, b1.reshape(1, -1), w2, b2.reshape(1, -1), w3, b3.reshape(1, -1))


def _dec_body(x_ref, w1, b1, w2, b2, w3, b3, o_ref):
    h = _ssp(_dot(x_ref[...], w1[...]) + b1[...])
    h = _ssp(_dot(h, w2[...]) + b2[...])
    o_ref[...] = _dot(h, w3[...]) + b3[...]


def _decode(x, params, tile):
    (w1, b1), (w2, b2), (w3, b3) = params
    n, din = x.shape
    dout = w3.shape[1]
    w3p = jnp.pad(w3, ((0, 0), (0, 8 - dout)))
    b3p = jnp.pad(b3, (0, 8 - dout))
    grid = (n + tile - 1) // tile
    out = pl.pallas_call(
        _dec_body,
        grid=(grid,),
        in_specs=[
            pl.BlockSpec((tile, din), lambda i: (i, 0)),
            _wspec(w1.shape), _wspec((1, w1.shape[1])),
            _wspec(w2.shape), _wspec((1, w2.shape[1])),
            _wspec((din * 0 + w2.shape[1], 8)), _wspec((1, 8)),
        ],
        out_specs=pl.BlockSpec((tile, 8), lambda i: (i, 0)),
        out_shape=jax.ShapeDtypeStruct((n, 8), jnp.float32),
    )(x, w1, b1.reshape(1, -1), w2, b2.reshape(1, -1), w3p, b3p.reshape(1, -1))
    return out[:, :dout]


def _sspb(x):
    xb = x.astype(jnp.bfloat16)
    zero = jnp.bfloat16(0.0)
    r = jnp.maximum(xb, zero) + jnp.log1p(jnp.exp(-jnp.abs(xb)))
    return r.astype(jnp.float32) - _LOG2


def _edge_body(e_ref, s_ref, r_ref, c_ref, gw, gb, w1e, w1s, w1r, b1, w2, b2,
               w3, b3, ne_ref, ms_ref, mr_ref):
    e = e_ref[...]
    h = _dot(e, w1e[...]) + _dot(s_ref[...], w1s[...]) + _dot(
        r_ref[...], w1r[...]) + b1[...]
    h = _sspb(h)
    h = _sspb(_dot(h, w2[...]) + b2[...])
    u = _ln(_dot(h, w3[...]) + b3[...])
    c = c_ref[...]
    g_r = jax.nn.sigmoid(
        jnp.sum(e * gw[0:1, :], axis=1, keepdims=True) + gb[0]) * c
    g_s = jax.nn.sigmoid(
        jnp.sum(e * gw[1:2, :], axis=1, keepdims=True) + gb[1]) * c
    ne_ref[...] = e + u
    ms_ref[...] = u * g_s
    mr_ref[...] = u * g_r


def _edge_step(edges, sent, recv, cutoff, p, tile):
    (w1, b1), (w2, b2), (w3, b3) = p["edge_mlp"]
    gw = jnp.concatenate([p["rw"], p["sw"]], axis=1).T
    gb = jnp.concatenate([p["rb"], p["sb"]])
    n = edges.shape[0]
    grid = (n + tile - 1) // tile
    row = lambda i: (i, 0)
    outs = pl.pallas_call(
        _edge_body,
        grid=(grid,),
        in_specs=[
            pl.BlockSpec((tile, LATENT), row),
            pl.BlockSpec((tile, LATENT), row),
            pl.BlockSpec((tile, LATENT), row),
            pl.BlockSpec((tile, 1), row),
            _wspec((2, LATENT)),
            pl.BlockSpec(memory_space=pltpu.SMEM),
            _wspec((LATENT, HIDDEN)), _wspec((LATENT, HIDDEN)),
            _wspec((LATENT, HIDDEN)), _wspec((1, HIDDEN)),
            _wspec((HIDDEN, HIDDEN)), _wspec((1, HIDDEN)),
            _wspec((HIDDEN, LATENT)), _wspec((1, LATENT)),
        ],
        out_specs=[
            pl.BlockSpec((tile, LATENT), row),
            pl.BlockSpec((tile, LATENT), row),
            pl.BlockSpec((tile, LATENT), row),
        ],
        out_shape=[jax.ShapeDtypeStruct((n, LATENT), jnp.float32)] * 3,
    )(edges, sent, recv, cutoff, gw, gb,
      w1[:LATENT], w1[LATENT:2 * LATENT], w1[2 * LATENT:], b1.reshape(1, -1),
      w2, b2.reshape(1, -1), w3, b3.reshape(1, -1))
    return outs


def _node_body(x_ref, ra0_ref, ra1_ref, sa0_ref, sa1_ref, w1n, w1r, w1s, b1,
               w2, b2, w3, b3, o_ref):
    x = x_ref[...]
    ra = ra0_ref[...] + ra1_ref[...]
    sa = sa0_ref[...] + sa1_ref[...]
    h = _dot(x, w1n[...]) + _dot(ra, w1r[...]) + _dot(sa, w1s[...]) + b1[...]
    h = _ssp(h)
    h = _ssp(_dot(h, w2[...]) + b2[...])
    u = _ln(_dot(h, w3[...]) + b3[...])
    o_ref[...] = x + u


def _node_step(nodes, aggs, p, tile):
    (w1, b1), (w2, b2), (w3, b3) = p["node_mlp"]
    (ra0, sa0), (ra1, sa1) = aggs
    n = nodes.shape[0]
    grid = (n + tile - 1) // tile
    row = lambda i: (i, 0)
    return pl.pallas_call(
        _node_body,
        grid=(grid,),
        in_specs=[
            pl.BlockSpec((tile, LATENT), row),
            pl.BlockSpec((tile, LATENT), row),
            pl.BlockSpec((tile, LATENT), row),
            pl.BlockSpec((tile, LATENT), row),
            pl.BlockSpec((tile, LATENT), row),
            _wspec((LATENT, HIDDEN)), _wspec((LATENT, HIDDEN)),
            _wspec((LATENT, HIDDEN)), _wspec((1, HIDDEN)),
            _wspec((HIDDEN, HIDDEN)), _wspec((1, HIDDEN)),
            _wspec((HIDDEN, LATENT)), _wspec((1, LATENT)),
        ],
        out_specs=pl.BlockSpec((tile, LATENT), row),
        out_shape=jax.ShapeDtypeStruct((n, LATENT), jnp.float32),
    )(nodes, ra0, ra1, sa0, sa1,
      w1[:LATENT], w1[LATENT:2 * LATENT], w1[2 * LATENT:], b1.reshape(1, -1),
      w2, b2.reshape(1, -1), w3, b3.reshape(1, -1))


_CHUNK = 128
_NSUB = 16


def _sc_gather(nodes, senders, receivers):
    ne = senders.shape[0]
    nchunks = ne // _CHUNK
    maxk = (nchunks + _NSUB - 1) // _NSUB
    mesh = plsc.VectorSubcoreMesh(core_axis_name="c", subcore_axis_name="s")

    @functools.partial(
        pl.kernel,
        out_type=(jax.ShapeDtypeStruct((ne, LATENT), jnp.float32),
                  jax.ShapeDtypeStruct((ne, LATENT), jnp.float32)),
        mesh=mesh,
        scratch_types=[
            pltpu.VMEM((_CHUNK,), jnp.int32),
            pltpu.VMEM((_CHUNK, LATENT), jnp.float32),
        ],
    )
    def k(nodes_hbm, snd_hbm, rcv_hbm, sent_hbm, recv_hbm, idx_v, buf_v):
        core = lax.axis_index("c")
        sub = lax.axis_index("s")

        def do(idx_hbm, out_hbm):
            @pl.loop(0, maxk)
            def _(kk):
                c = sub + kk * _NSUB

                @pl.when(c < nchunks)
                def _():
                    off = pl.multiple_of(c * _CHUNK, 8)
                    pltpu.sync_copy(idx_hbm.at[pl.ds(off, _CHUNK)], idx_v)
                    pltpu.sync_copy(nodes_hbm.at[idx_v], buf_v)
                    pltpu.sync_copy(buf_v, out_hbm.at[pl.ds(off, _CHUNK)])

        @pl.when(core == 0)
        def _():
            do(rcv_hbm, recv_hbm)

        @pl.when(core == 1)
        def _():
            do(snd_hbm, sent_hbm)

    sent, recv = k(nodes, senders, receivers)
    return sent, recv


def _sc_segsum(msg_r, msg_s, receivers, senders, zeros):
    ne = receivers.shape[0]
    nchunks = ne // _CHUNK
    maxk = (nchunks + _NSUB - 1) // _NSUB
    mesh = plsc.VectorSubcoreMesh(core_axis_name="c", subcore_axis_name="s")
    rchunk = 80
    nrch = N_NODES // rchunk
    maxr = (nrch + _NSUB - 1) // _NSUB

    @functools.partial(
        pl.kernel,
        out_type=jax.ShapeDtypeStruct((2, N_NODES, LATENT), jnp.float32),
        mesh=mesh,
        scratch_types=[
            pltpu.VMEM((_CHUNK,), jnp.int32),
            pltpu.VMEM((_CHUNK, LATENT), jnp.float32),
            pltpu.VMEM_SHARED((N_NODES, LATENT), jnp.float32),
        ],
    )
    def k(mr_hbm, ms_hbm, rcv_hbm, snd_hbm, z_hbm, out_hbm, idx_v, buf_v, acc):
        core = lax.axis_index("c")
        sub = lax.axis_index("s")

        @pl.loop(0, maxr)
        def _(kk):
            c = sub + kk * _NSUB

            @pl.when(c < nrch)
            def _():
                off = pl.multiple_of(c * rchunk, 8)
                pltpu.sync_copy(z_hbm.at[pl.ds(off, rchunk)],
                                acc.at[pl.ds(off, rchunk)])

        plsc.subcore_barrier()

        def do(m_hbm, i_hbm):
            @pl.loop(0, maxk)
            def _(kk):
                c = sub + kk * _NSUB

                @pl.when(c < nchunks)
                def _():
                    off = pl.multiple_of(c * _CHUNK, 8)
                    pltpu.sync_copy(i_hbm.at[pl.ds(off, _CHUNK)], idx_v)
                    pltpu.sync_copy(m_hbm.at[pl.ds(off, _CHUNK)], buf_v)
                    pltpu.sync_copy(buf_v, acc.at[idx_v], add=True)

        @pl.when(core == 0)
        def _():
            do(mr_hbm, rcv_hbm)

        @pl.when(core == 1)
        def _():
            do(ms_hbm, snd_hbm)

        plsc.subcore_barrier()

        @pl.loop(0, maxr)
        def _(kk):
            c = sub + kk * _NSUB

            @pl.when(c < nrch)
            def _():
                off = pl.multiple_of(c * rchunk, 8)
                pltpu.sync_copy(acc.at[pl.ds(off, rchunk)],
                                out_hbm.at[core, pl.ds(off, rchunk)])

    out = k(msg_r, msg_s, receivers, senders, zeros)
    return out[0], out[1]


def kernel(node_features, edge_features, senders, receivers, cutoff, params):
    nodes = _mlp_ln(node_features, params["enc_node"], 512)
    zeros = jnp.zeros((N_NODES, LATENT), jnp.float32)
    h = N_EDGES // 2
    sl = (slice(0, h), slice(h, N_EDGES))
    snd = [senders[q] for q in sl]
    rcv = [receivers[q] for q in sl]
    cut = [cutoff[q] for q in sl]
    edges = [_mlp_ln(edge_features[q], params["enc_edge"], 512) for q in sl]
    for s in range(3):
        p = params["gnn"][s]
        new_e, aggs = [], []
        gathered = [_sc_gather(nodes, snd[i], rcv[i]) for i in range(2)]
        for i in range(2):
            sent_i, recv_i = gathered[i]
            ne, ms, mr = _edge_step(edges[i], sent_i, recv_i, cut[i], p, 512)
            new_e.append(ne)
            aggs.append(_sc_segsum(mr, ms, rcv[i], snd[i], zeros))
        nodes = _node_step(nodes, aggs, p, 512)
        edges = new_e
    return _decode(nodes, params["dec"], 512)
